# baseline (speedup 1.0000x reference)
import functools

import jax
import jax.numpy as jnp
from jax import lax
from jax.experimental import pallas as pl
from jax.experimental.pallas import tpu as pltpu
from jax.experimental.pallas import tpu_sc as plsc

_VOCAB = 100000
_EMB = 128
_HID = 64
_OUT = 3
_B = 4096
_L = 200
_LANES = 16
_NVREG = _EMB // _LANES
_SW = 128
_NBUF = 5
_SUP = 25


@functools.lru_cache(maxsize=None)
def _make_pool_kernel():
    info = plsc.get_sparse_core_info()
    nc, ns = info.num_cores, info.num_subcores
    nw = nc * ns
    bpw = _B // nw
    nstream = bpw * _L // _SW
    nsup = nstream // _SUP
    spsup = _SUP * _SW // _L

    mesh = plsc.VectorSubcoreMesh(core_axis_name="c", subcore_axis_name="s")

    @functools.partial(
        pl.kernel,
        mesh=mesh,
        out_type=jax.ShapeDtypeStruct((_B, _EMB), jnp.float32),
        scratch_types=[
            pltpu.VMEM((nstream, _SW), jnp.int32),
            pltpu.VMEM((_NBUF, _SW, _EMB), jnp.float32),
            pltpu.VMEM((bpw, _EMB), jnp.float32),
        ] + [pltpu.SemaphoreType.DMA] * _NBUF,
    )
    def pool(x_hbm, table_hbm, out_hbm, idx_v, rows_v, acc_v, *sems):
        wid = lax.axis_index("s") * nc + lax.axis_index("c")
        pltpu.sync_copy(x_hbm.at[wid], idx_v)

        def issue(r, buf):
            pltpu.async_copy(table_hbm.at[idx_v.at[r]], rows_v.at[buf], sems[buf])

        def wait_buf(buf):
            pltpu.make_async_copy(
                table_hbm.at[pl.ds(0, _SW)], rows_v.at[buf], sems[buf]
            ).wait()

        def seg_sum(buf, lo, length, acc):
            def row_body(i, a):
                for u in range(8):
                    r = lo + i * 8 + u
                    a = tuple(
                        a[c] + rows_v[buf, r, pl.ds(_LANES * c, _LANES)]
                        for c in range(_NVREG)
                    )
                return a

            return lax.fori_loop(0, length // 8, row_body, acc)

        def store(s, acc):
            for c in range(_NVREG):
                acc_v[s, pl.ds(_LANES * c, _LANES)] = acc[c]

        zeros = tuple(jnp.zeros((_LANES,), jnp.float32) for _ in range(_NVREG))

        for b in range(_NBUF):
            issue(b, b)

        def super_body(i, carry):
            s0 = spsup * i
            g0 = _SUP * i
            cnt = 0
            acc = zeros
            for t in range(_SUP):
                buf = t % _NBUF
                wait_buf(buf)
                m = (_SW * t) % _L
                b = _L - m
                if b < _SW:
                    acc = seg_sum(buf, 0, b, acc)
                    store(s0 + cnt, acc)
                    cnt += 1
                    acc = seg_sum(buf, b, _SW - b, zeros)
                else:
                    acc = seg_sum(buf, 0, _SW, acc)
                    if b == _SW:
                        store(s0 + cnt, acc)
                        cnt += 1
                        acc = zeros
                if t < _SUP - _NBUF:
                    issue(g0 + t + _NBUF, buf)
                else:
                    nxt = g0 + t + _NBUF

                    @pl.when(nxt < nstream)
                    def _():
                        issue(nxt, buf)
            return carry

        lax.fori_loop(0, nsup, super_body, 0)
        pltpu.sync_copy(acc_v, out_hbm.at[pl.ds(wid * bpw, bpw)])

    return pool, nw


def _mlp_body(p_ref, w1_ref, b1_ref, w2_ref, b2_ref, w3_ref, b3_ref, o_ref):
    h = p_ref[...] * (1.0 / _L)
    h = jnp.maximum(
        jnp.dot(h, w1_ref[...], preferred_element_type=jnp.float32) + b1_ref[...],
        0.0,
    )
    h = jnp.maximum(
        jnp.dot(h, w2_ref[...], preferred_element_type=jnp.float32) + b2_ref[...],
        0.0,
    )
    o_ref[...] = (
        jnp.dot(h, w3_ref[...], preferred_element_type=jnp.float32) + b3_ref[...]
    )


def kernel(x, table, W1, b1, W2, b2, W3, b3):
    pool, nw = _make_pool_kernel()
    xr = x.astype(jnp.int32).reshape(nw, _B // nw * _L // _SW, _SW)
    pooled = pool(xr, table)
    out = pl.pallas_call(
        _mlp_body,
        out_shape=jax.ShapeDtypeStruct((_B, _OUT), jnp.float32),
    )(
        pooled,
        W1,
        b1.reshape(1, -1),
        W2,
        b2.reshape(1, -1),
        W3,
        b3.reshape(1, -1),
    )
    return out


# speedup vs baseline: 1.0600x; 1.0600x over previous
import functools

import jax
import jax.numpy as jnp
from jax import lax
from jax.experimental import pallas as pl
from jax.experimental.pallas import tpu as pltpu
from jax.experimental.pallas import tpu_sc as plsc

_VOCAB = 100000
_EMB = 128
_HID = 64
_OUT = 3
_B = 4096
_L = 200
_NCHUNK = 2
_CHUNK = _L // _NCHUNK
_NBUF = 3
_LANES = 16
_NVREG = _EMB // _LANES
_UNROLL = 10


@functools.lru_cache(maxsize=None)
def _make_pool_kernel():
    info = plsc.get_sparse_core_info()
    nc, ns = info.num_cores, info.num_subcores
    nw = nc * ns
    bpw = _B // nw

    mesh = plsc.VectorSubcoreMesh(core_axis_name="c", subcore_axis_name="s")

    @functools.partial(
        pl.kernel,
        mesh=mesh,
        out_type=jax.ShapeDtypeStruct((_B, _EMB), jnp.float32),
        scratch_types=[
            pltpu.VMEM((bpw, _NCHUNK, _CHUNK), jnp.int32),
            pltpu.VMEM((_NBUF, _L, _EMB), jnp.float32),
            pltpu.VMEM((bpw, _EMB), jnp.float32),
        ] + [pltpu.SemaphoreType.DMA] * _NBUF,
    )
    def pool(x_hbm, table_hbm, out_hbm, idx_v, rows_v, acc_v, *sems):
        wid = lax.axis_index("s") * nc + lax.axis_index("c")
        base = wid * bpw
        pltpu.sync_copy(x_hbm.at[pl.ds(base, bpw)], idx_v)

        def issue(s, buf):
            for j in range(_NCHUNK):
                pltpu.async_copy(
                    table_hbm.at[idx_v.at[s, j]],
                    rows_v.at[buf, pl.ds(j * _CHUNK, _CHUNK)],
                    sems[buf],
                )

        def wait_buf(buf):
            pltpu.make_async_copy(
                table_hbm.at[pl.ds(0, _L)], rows_v.at[buf], sems[buf]
            ).wait()

        def issue_one(s, j, buf):
            pltpu.async_copy(
                table_hbm.at[idx_v.at[s, j]],
                rows_v.at[buf, pl.ds(j * _CHUNK, _CHUNK)],
                sems[buf],
            )

        def half_sum(buf, j, acc):
            def row_body(i, a):
                for u in range(_UNROLL):
                    r = j * _CHUNK + i * _UNROLL + u
                    a = tuple(
                        a[c] + rows_v[buf, r, pl.ds(_LANES * c, _LANES)]
                        for c in range(_NVREG)
                    )
                return a

            return lax.fori_loop(0, _CHUNK // _UNROLL, row_body, acc)

        def store(s, acc):
            for c in range(_NVREG):
                acc_v[s, pl.ds(_LANES * c, _LANES)] = acc[c]

        zeros = tuple(jnp.zeros((_LANES,), jnp.float32) for _ in range(_NVREG))

        def process(s, buf, nxt):
            wait_buf(buf)
            acc = half_sum(buf, 0, zeros)
            if nxt is not None:
                issue_one(nxt, 0, buf)
            acc = half_sum(buf, 1, acc)
            store(s, acc)
            if nxt is not None:
                issue_one(nxt, 1, buf)

        for b in range(_NBUF):
            issue(b, b)

        nsup = (bpw - _NBUF) // _NBUF

        def super_body(i, carry):
            s0 = _NBUF * i
            for k in range(_NBUF):
                process(s0 + k, k, s0 + k + _NBUF)
            return carry

        lax.fori_loop(0, nsup, super_body, 0)

        rem = bpw - nsup * _NBUF
        for t in range(rem):
            s = bpw - rem + t
            process(s, t % _NBUF, s + _NBUF if s + _NBUF < bpw else None)

        pltpu.sync_copy(acc_v, out_hbm.at[pl.ds(base, bpw)])

    return pool


def _mlp_body(p_ref, w1_ref, b1_ref, w2_ref, b2_ref, w3_ref, b3_ref, o_ref):
    h = p_ref[...] * (1.0 / _L)
    h = jnp.maximum(
        jnp.dot(h, w1_ref[...], preferred_element_type=jnp.float32) + b1_ref[...],
        0.0,
    )
    h = jnp.maximum(
        jnp.dot(h, w2_ref[...], preferred_element_type=jnp.float32) + b2_ref[...],
        0.0,
    )
    o_ref[...] = (
        jnp.dot(h, w3_ref[...], preferred_element_type=jnp.float32) + b3_ref[...]
    )


def kernel(x, table, W1, b1, W2, b2, W3, b3):
    x3 = x.astype(jnp.int32).reshape(_B, _NCHUNK, _CHUNK)
    pooled = _make_pool_kernel()(x3, table)
    out = pl.pallas_call(
        _mlp_body,
        out_shape=jax.ShapeDtypeStruct((_B, _OUT), jnp.float32),
    )(
        pooled,
        W1,
        b1.reshape(1, -1),
        W2,
        b2.reshape(1, -1),
        W3,
        b3.reshape(1, -1),
    )
    return out


# speedup vs baseline: 1.0690x; 1.0084x over previous
import functools

import jax
import jax.numpy as jnp
from jax import lax
from jax.experimental import pallas as pl
from jax.experimental.pallas import tpu as pltpu
from jax.experimental.pallas import tpu_sc as plsc

_VOCAB = 100000
_EMB = 128
_HID = 64
_OUT = 3
_B = 4096
_L = 200
_NCHUNK = 2
_CHUNK = _L // _NCHUNK
_NBUF = 3
_LANES = 16
_NVREG = _EMB // _LANES
_UNROLL = 8


@functools.lru_cache(maxsize=None)
def _make_pool_kernel():
    info = plsc.get_sparse_core_info()
    nc, ns = info.num_cores, info.num_subcores
    nw = nc * ns
    bpw = _B // nw

    mesh = plsc.VectorSubcoreMesh(core_axis_name="c", subcore_axis_name="s")

    @functools.partial(
        pl.kernel,
        mesh=mesh,
        out_type=jax.ShapeDtypeStruct((_B, _EMB), jnp.float32),
        scratch_types=[
            pltpu.VMEM((bpw, _NCHUNK, _CHUNK), jnp.int32),
            pltpu.VMEM((_NBUF, _L, _EMB), jnp.float32),
            pltpu.VMEM((bpw, _EMB), jnp.float32),
        ] + [pltpu.SemaphoreType.DMA] * _NBUF,
    )
    def pool(x_hbm, table_hbm, out_hbm, idx_v, rows_v, acc_v, *sems):
        wid = lax.axis_index("s") * nc + lax.axis_index("c")
        base = wid * bpw
        pltpu.sync_copy(x_hbm.at[pl.ds(base, bpw)], idx_v)

        def issue(s, buf):
            for j in range(_NCHUNK):
                pltpu.async_copy(
                    table_hbm.at[idx_v.at[s, j]],
                    rows_v.at[buf, pl.ds(j * _CHUNK, _CHUNK)],
                    sems[buf],
                )

        def wait_buf(buf):
            pltpu.make_async_copy(
                table_hbm.at[pl.ds(0, _L)], rows_v.at[buf], sems[buf]
            ).wait()

        def accumulate(s, buf):
            def row_body(i, a):
                for u in range(_UNROLL):
                    r = i * _UNROLL + u
                    a = tuple(
                        a[c] + rows_v[buf, r, pl.ds(_LANES * c, _LANES)]
                        for c in range(_NVREG)
                    )
                return a

            acc = lax.fori_loop(
                0, _L // _UNROLL, row_body,
                tuple(jnp.zeros((_LANES,), jnp.float32) for _ in range(_NVREG)),
            )
            for c in range(_NVREG):
                acc_v[s, pl.ds(_LANES * c, _LANES)] = acc[c]

        for b in range(_NBUF):
            issue(b, b)

        nsup = (bpw - _NBUF) // _NBUF

        def super_body(i, carry):
            s0 = _NBUF * i
            for k in range(_NBUF):
                s = s0 + k
                wait_buf(k)
                accumulate(s, k)
                issue(s + _NBUF, k)
            return carry

        lax.fori_loop(0, nsup, super_body, 0)

        rem = bpw - nsup * _NBUF
        for t in range(rem):
            s = bpw - rem + t
            buf = t % _NBUF
            wait_buf(buf)
            accumulate(s, buf)
            if s + _NBUF < bpw:
                issue(s + _NBUF, buf)

        pltpu.sync_copy(acc_v, out_hbm.at[pl.ds(base, bpw)])

    return pool


def _mlp_body(p_ref, w1_ref, b1_ref, w2_ref, b2_ref, w3_ref, b3_ref, o_ref):
    h = p_ref[...] * (1.0 / _L)
    h = jnp.maximum(
        jnp.dot(h, w1_ref[...], preferred_element_type=jnp.float32) + b1_ref[...],
        0.0,
    )
    h = jnp.maximum(
        jnp.dot(h, w2_ref[...], preferred_element_type=jnp.float32) + b2_ref[...],
        0.0,
    )
    o_ref[...] = (
        jnp.dot(h, w3_ref[...], preferred_element_type=jnp.float32) + b3_ref[...]
    )


def kernel(x, table, W1, b1, W2, b2, W3, b3):
    x3 = x.astype(jnp.int32).reshape(_B, _NCHUNK, _CHUNK)
    pooled = _make_pool_kernel()(x3, table)
    out = pl.pallas_call(
        _mlp_body,
        out_shape=jax.ShapeDtypeStruct((_B, _OUT), jnp.float32),
    )(
        pooled,
        W1,
        b1.reshape(1, -1),
        W2,
        b2.reshape(1, -1),
        W3,
        b3.reshape(1, -1),
    )
    return out
